# baseline (device time: 90021 ns/iter reference)
import jax
import jax.numpy as jnp
from jax import lax
from jax.experimental import pallas as pl
from jax.experimental.pallas import tpu as pltpu

N_DEV = 8

GROUPS = (
    ((2, 1, 0), 640),
    ((1, 0, 2), 640),
    ((0, 2, 1), 768),
)


def kernel(x, w_mat, scale_x, scale_w):
    m, k_per = x.shape
    _, n = w_mat.shape
    m_per = m // N_DEV
    assert sum(w for _, w in GROUPS) == n

    def body(x_ref, w_ref, sx_ref, sw_ref, out_ref, *scratch):
        sc = [scratch[9 * g:9 * (g + 1)] for g in range(len(GROUPS))]

        my = lax.axis_index("i")
        p4 = lax.rem(my, 4)
        mbit = (
            lax.rem(lax.rem(p4, 2) + p4 // 2, 2),
            p4 // 2,
            my // 4,
        )

        def pos_from_bits(bx, by, bz):
            return 4 * bz + bx + by * (3 - 2 * bx)

        def partner(axis):
            b = list(mbit)
            b[axis] = 1 - b[axis]
            return pos_from_bits(*b)

        def chunk_c(bits_by_axis):
            return pos_from_bits(bits_by_axis[0], bits_by_axis[1], bits_by_axis[2])

        barrier = pltpu.get_barrier_semaphore()
        for a in range(3):
            pl.semaphore_signal(
                barrier, inc=1, device_id=(partner(a),),
                device_id_type=pl.DeviceIdType.MESH,
            )
        pl.semaphore_wait(barrier, 3)

        col0s = []
        c0 = 0
        for (_, w) in GROUPS:
            col0s.append(c0)
            c0 += w

        def partial(c, g):
            xb = x_ref[pl.ds(c * m_per, m_per), :]
            wb = w_ref[:, col0s[g]:col0s[g] + GROUPS[g][1]]
            return jnp.dot(xb, wb, preferred_element_type=jnp.float32)

        def mk(src, dst, sems, recvs, stage, axis):
            return pltpu.make_async_remote_copy(
                src_ref=src, dst_ref=dst,
                send_sem=sems.at[stage], recv_sem=recvs.at[stage],
                device_id=(partner(axis),),
                device_id_type=pl.DeviceIdType.MESH,
            )

        def slot_bits(axes, b1, j):
            bits = [None, None, None]
            bits[axes[0]] = b1
            bits[axes[1]] = j // 2
            bits[axes[2]] = j % 2
            return bits

        order = sorted(range(len(GROUPS)), key=lambda g: -GROUPS[g][1])

        def gbits(g):
            axes = GROUPS[g][0]
            return mbit[axes[0]], mbit[axes[1]], mbit[axes[2]]

        rd1, rd2, rd3 = {}, {}, {}
        for g in order:
            axes, w = GROUPS[g]
            sb1, rb1, ss, rs = sc[g][1], sc[g][2], sc[g][7], sc[g][8]
            m1, _, _ = gbits(g)
            for j in range(4):
                c = chunk_c(slot_bits(axes, 1 - m1, j))
                sb1[j] = partial(c, g).astype(jnp.bfloat16)
            rd1[g] = mk(sb1, rb1, ss, rs, 0, axes[0])
            rd1[g].start()
        for g in order:
            axes, w = GROUPS[g]
            acc = sc[g][0]
            m1, m2, _ = gbits(g)
            for j2 in range(2):
                c = chunk_c(slot_bits(axes, m1, 2 * (1 - m2) + j2))
                acc[pl.ds(2 * (1 - m2) + j2, 1)] = partial(c, g)[None]
        for g in order:
            axes, w = GROUPS[g]
            acc, rb1, sb2, rb2, ss, rs = (
                sc[g][0], sc[g][2], sc[g][3], sc[g][4], sc[g][7], sc[g][8])
            _, m2, _ = gbits(g)
            rd1[g].wait()
            snd = pl.ds(2 * (1 - m2), 2)
            sb2[:, :, :] = (acc[snd] + rb1[snd].astype(jnp.float32)).astype(jnp.bfloat16)
            rd2[g] = mk(sb2, rb2, ss, rs, 1, axes[1])
            rd2[g].start()
        for g in order:
            axes, w = GROUPS[g]
            acc, rb1 = sc[g][0], sc[g][2]
            m1, m2, _ = gbits(g)
            for j2 in range(2):
                slot = pl.ds(2 * m2 + j2, 1)
                c = chunk_c(slot_bits(axes, m1, 2 * m2 + j2))
                acc[slot] = partial(c, g)[None] + rb1[slot].astype(jnp.float32)
        for g in order:
            axes, w = GROUPS[g]
            acc, rb2, sb3, rb3, ss, rs = (
                sc[g][0], sc[g][4], sc[g][5], sc[g][6], sc[g][7], sc[g][8])
            _, m2, m3 = gbits(g)
            rd2[g].wait()
            sb3[:, :, :] = (
                acc[pl.ds(2 * m2 + (1 - m3), 1)]
                + rb2[pl.ds(1 - m3, 1)].astype(jnp.float32)
            ).astype(jnp.bfloat16)
            rd3[g] = mk(sb3, rb3, ss, rs, 2, axes[2])
            rd3[g].start()
        scale = sx_ref[0] * sw_ref[0]
        for g in order:
            axes, w = GROUPS[g]
            acc, rb2, rb3 = sc[g][0], sc[g][4], sc[g][6]
            _, m2, m3 = gbits(g)
            rd3[g].wait()
            final = (
                acc[pl.ds(2 * m2 + m3, 1)][0]
                + rb2[pl.ds(m3, 1)][0].astype(jnp.float32)
                + rb3[0].astype(jnp.float32)
            )
            y = final * scale
            out_ref[:, col0s[g]:col0s[g] + GROUPS[g][1]] = y * jax.nn.sigmoid(y)

    scratch = []
    for (_, w) in GROUPS:
        scratch += [
            pltpu.VMEM((4, m_per, w), jnp.float32),
            pltpu.VMEM((4, m_per, w), jnp.bfloat16),
            pltpu.VMEM((4, m_per, w), jnp.bfloat16),
            pltpu.VMEM((2, m_per, w), jnp.bfloat16),
            pltpu.VMEM((2, m_per, w), jnp.bfloat16),
            pltpu.VMEM((1, m_per, w), jnp.bfloat16),
            pltpu.VMEM((1, m_per, w), jnp.bfloat16),
            pltpu.SemaphoreType.DMA((3,)),
            pltpu.SemaphoreType.DMA((3,)),
        ]

    return pl.pallas_call(
        body,
        out_shape=jax.ShapeDtypeStruct((m_per, n), jnp.float32),
        in_specs=[
            pl.BlockSpec(memory_space=pltpu.VMEM),
            pl.BlockSpec(memory_space=pltpu.VMEM),
            pl.BlockSpec(memory_space=pltpu.SMEM),
            pl.BlockSpec(memory_space=pltpu.SMEM),
        ],
        out_specs=pl.BlockSpec(memory_space=pltpu.VMEM),
        scratch_shapes=scratch,
        compiler_params=pltpu.CompilerParams(
            collective_id=0,
            vmem_limit_bytes=110 * 1024 * 1024,
        ),
    )(x.astype(jnp.bfloat16), w_mat.astype(jnp.bfloat16), scale_x, scale_w)


# device time: 86689 ns/iter; 1.0384x vs baseline; 1.0384x over previous
import jax
import jax.numpy as jnp
from jax import lax
from jax.experimental import pallas as pl
from jax.experimental.pallas import tpu as pltpu

N_DEV = 8

GROUPS = (
    ((2, 1, 0), 640),
    ((1, 0, 2), 640),
    ((0, 2, 1), 768),
)


def kernel(x, w_mat, scale_x, scale_w):
    m, k_per = x.shape
    _, n = w_mat.shape
    m_per = m // N_DEV
    assert sum(w for _, w in GROUPS) == n

    def body(x_ref, w_ref, sx_ref, sw_ref, out_ref, *scratch):
        sc = [scratch[9 * g:9 * (g + 1)] for g in range(len(GROUPS))]

        my = lax.axis_index("i")
        p4 = lax.rem(my, 4)
        mbit = (
            lax.rem(lax.rem(p4, 2) + p4 // 2, 2),
            p4 // 2,
            my // 4,
        )

        def pos_from_bits(bx, by, bz):
            return 4 * bz + bx + by * (3 - 2 * bx)

        def partner(axis):
            b = list(mbit)
            b[axis] = 1 - b[axis]
            return pos_from_bits(*b)

        def chunk_c(bits_by_axis):
            return pos_from_bits(bits_by_axis[0], bits_by_axis[1], bits_by_axis[2])

        barrier = pltpu.get_barrier_semaphore()
        for a in range(3):
            pl.semaphore_signal(
                barrier, inc=1, device_id=(partner(a),),
                device_id_type=pl.DeviceIdType.MESH,
            )
        pl.semaphore_wait(barrier, 3)

        col0s = []
        c0 = 0
        for (_, w) in GROUPS:
            col0s.append(c0)
            c0 += w

        def partial(c, g):
            xb = x_ref[pl.ds(c * m_per, m_per), :]
            wb = w_ref[:, col0s[g]:col0s[g] + GROUPS[g][1]].astype(jnp.bfloat16)
            return jnp.dot(xb, wb, preferred_element_type=jnp.float32)

        def mk(src, dst, sems, recvs, stage, axis):
            return pltpu.make_async_remote_copy(
                src_ref=src, dst_ref=dst,
                send_sem=sems.at[stage], recv_sem=recvs.at[stage],
                device_id=(partner(axis),),
                device_id_type=pl.DeviceIdType.MESH,
            )

        def slot_bits(axes, b1, j):
            bits = [None, None, None]
            bits[axes[0]] = b1
            bits[axes[1]] = j // 2
            bits[axes[2]] = j % 2
            return bits

        order = sorted(range(len(GROUPS)), key=lambda g: -GROUPS[g][1])

        def gbits(g):
            axes = GROUPS[g][0]
            return mbit[axes[0]], mbit[axes[1]], mbit[axes[2]]

        rd1, rd2, rd3 = {}, {}, {}
        for g in order:
            axes, w = GROUPS[g]
            sb1, rb1, ss, rs = sc[g][1], sc[g][2], sc[g][7], sc[g][8]
            m1, _, _ = gbits(g)
            for j in range(4):
                c = chunk_c(slot_bits(axes, 1 - m1, j))
                sb1[j] = partial(c, g).astype(jnp.bfloat16)
            rd1[g] = mk(sb1, rb1, ss, rs, 0, axes[0])
            rd1[g].start()
        for g in order:
            axes, w = GROUPS[g]
            acc = sc[g][0]
            m1, m2, _ = gbits(g)
            for j2 in range(2):
                c = chunk_c(slot_bits(axes, m1, 2 * (1 - m2) + j2))
                acc[pl.ds(2 * (1 - m2) + j2, 1)] = partial(c, g)[None]
        for g in order:
            axes, w = GROUPS[g]
            acc, rb1, sb2, rb2, ss, rs = (
                sc[g][0], sc[g][2], sc[g][3], sc[g][4], sc[g][7], sc[g][8])
            _, m2, _ = gbits(g)
            rd1[g].wait()
            snd = pl.ds(2 * (1 - m2), 2)
            sb2[:, :, :] = (acc[snd] + rb1[snd].astype(jnp.float32)).astype(jnp.bfloat16)
            rd2[g] = mk(sb2, rb2, ss, rs, 1, axes[1])
            rd2[g].start()
        for g in order:
            axes, w = GROUPS[g]
            acc, rb1 = sc[g][0], sc[g][2]
            m1, m2, _ = gbits(g)
            for j2 in range(2):
                slot = pl.ds(2 * m2 + j2, 1)
                c = chunk_c(slot_bits(axes, m1, 2 * m2 + j2))
                acc[slot] = partial(c, g)[None] + rb1[slot].astype(jnp.float32)
        for g in order:
            axes, w = GROUPS[g]
            acc, rb2, sb3, rb3, ss, rs = (
                sc[g][0], sc[g][4], sc[g][5], sc[g][6], sc[g][7], sc[g][8])
            _, m2, m3 = gbits(g)
            rd2[g].wait()
            sb3[:, :, :] = (
                acc[pl.ds(2 * m2 + (1 - m3), 1)]
                + rb2[pl.ds(1 - m3, 1)].astype(jnp.float32)
            ).astype(jnp.bfloat16)
            rd3[g] = mk(sb3, rb3, ss, rs, 2, axes[2])
            rd3[g].start()
        scale = sx_ref[0] * sw_ref[0]
        for g in order:
            axes, w = GROUPS[g]
            acc, rb2, rb3 = sc[g][0], sc[g][4], sc[g][6]
            _, m2, m3 = gbits(g)
            rd3[g].wait()
            final = (
                acc[pl.ds(2 * m2 + m3, 1)][0]
                + rb2[pl.ds(m3, 1)][0].astype(jnp.float32)
                + rb3[0].astype(jnp.float32)
            )
            y = final * scale
            out_ref[:, col0s[g]:col0s[g] + GROUPS[g][1]] = y * jax.nn.sigmoid(y)

    scratch = []
    for (_, w) in GROUPS:
        scratch += [
            pltpu.VMEM((4, m_per, w), jnp.float32),
            pltpu.VMEM((4, m_per, w), jnp.bfloat16),
            pltpu.VMEM((4, m_per, w), jnp.bfloat16),
            pltpu.VMEM((2, m_per, w), jnp.bfloat16),
            pltpu.VMEM((2, m_per, w), jnp.bfloat16),
            pltpu.VMEM((1, m_per, w), jnp.bfloat16),
            pltpu.VMEM((1, m_per, w), jnp.bfloat16),
            pltpu.SemaphoreType.DMA((3,)),
            pltpu.SemaphoreType.DMA((3,)),
        ]

    return pl.pallas_call(
        body,
        out_shape=jax.ShapeDtypeStruct((m_per, n), jnp.float32),
        in_specs=[
            pl.BlockSpec(memory_space=pltpu.VMEM),
            pl.BlockSpec(memory_space=pltpu.VMEM),
            pl.BlockSpec(memory_space=pltpu.SMEM),
            pl.BlockSpec(memory_space=pltpu.SMEM),
        ],
        out_specs=pl.BlockSpec(memory_space=pltpu.VMEM),
        scratch_shapes=scratch,
        compiler_params=pltpu.CompilerParams(
            collective_id=0,
            vmem_limit_bytes=110 * 1024 * 1024,
        ),
    )(x.astype(jnp.bfloat16), w_mat, scale_x, scale_w)


# device time: 75694 ns/iter; 1.1893x vs baseline; 1.1453x over previous
import jax
import jax.numpy as jnp
from jax import lax
from jax.experimental import pallas as pl
from jax.experimental.pallas import tpu as pltpu

N_DEV = 8

GROUPS = (
    ((2, 1, 0), 640),
    ((1, 0, 2), 640),
    ((0, 2, 1), 768),
)
ORDER = sorted(range(len(GROUPS)), key=lambda g: -GROUPS[g][1])


def kernel(x, w_mat, scale_x, scale_w):
    m, k_per = x.shape
    _, n = w_mat.shape
    m_per = m // N_DEV
    assert sum(w for _, w in GROUPS) == n

    def body(x_ref, w_ref, sx_ref, sw_ref, out_ref, *scratch):
        sc = [scratch[9 * g:9 * (g + 1)] for g in range(len(GROUPS))]

        my = lax.axis_index("i")
        p4 = lax.rem(my, 4)
        mbit = (
            lax.rem(lax.rem(p4, 2) + p4 // 2, 2),
            p4 // 2,
            my // 4,
        )

        def pos_from_bits(bx, by, bz):
            return 4 * bz + bx + by * (3 - 2 * bx)

        def partner(axis):
            b = list(mbit)
            b[axis] = 1 - b[axis]
            return pos_from_bits(*b)

        def bxor(j, mb):
            return 1 - mb if j else mb

        barrier = pltpu.get_barrier_semaphore()
        for a in range(3):
            pl.semaphore_signal(
                barrier, inc=1, device_id=(partner(a),),
                device_id_type=pl.DeviceIdType.MESH,
            )
        pl.semaphore_wait(barrier, 3)

        col0s = []
        c0 = 0
        for (_, w) in GROUPS:
            col0s.append(c0)
            c0 += w

        def partial(c, g):
            xb = x_ref[pl.ds(c * m_per, m_per), :].astype(jnp.bfloat16)
            wb = w_ref[:, col0s[g]:col0s[g] + GROUPS[g][1]].astype(jnp.bfloat16)
            return jnp.dot(xb, wb, preferred_element_type=jnp.float32)

        def mk(src, dst, g, stage, axis):
            ss, rs = sc[g][7], sc[g][8]
            return pltpu.make_async_remote_copy(
                src_ref=src, dst_ref=dst,
                send_sem=ss.at[stage], recv_sem=rs.at[stage],
                device_id=(partner(axis),),
                device_id_type=pl.DeviceIdType.MESH,
            )

        def slot_chunk(g, b1, j):
            axes = GROUPS[g][0]
            bits = [None, None, None]
            bits[axes[0]] = b1
            bits[axes[1]] = bxor(j // 2, mbit[axes[1]])
            bits[axes[2]] = bxor(j % 2, mbit[axes[2]])
            return pos_from_bits(bits[0], bits[1], bits[2])

        rd1a, rd1b, rd2, rd3 = {}, {}, {}, {}
        for g in ORDER:
            axes, w = GROUPS[g]
            sb1, rb1 = sc[g][1], sc[g][2]
            m1 = mbit[axes[0]]
            for j in (2, 3):
                sb1[j] = partial(slot_chunk(g, 1 - m1, j), g).astype(jnp.bfloat16)
            rd1a[g] = mk(sb1.at[pl.ds(2, 2)], rb1.at[pl.ds(2, 2)], g, 0, axes[0])
            rd1a[g].start()
        for g in ORDER:
            axes, w = GROUPS[g]
            sb1, rb1 = sc[g][1], sc[g][2]
            m1 = mbit[axes[0]]
            for j in (0, 1):
                sb1[j] = partial(slot_chunk(g, 1 - m1, j), g).astype(jnp.bfloat16)
            rd1b[g] = mk(sb1.at[pl.ds(0, 2)], rb1.at[pl.ds(0, 2)], g, 1, axes[0])
            rd1b[g].start()
        for g in ORDER:
            acc = sc[g][0]
            m1 = mbit[GROUPS[g][0][0]]
            for j in (2, 3):
                acc[j] = partial(slot_chunk(g, m1, j), g)
        for g in ORDER:
            axes, w = GROUPS[g]
            acc, rb1, sb2, rb2 = sc[g][0], sc[g][2], sc[g][3], sc[g][4]
            rd1a[g].wait()
            sb2[:, :, :] = (
                acc[pl.ds(2, 2)] + rb1[pl.ds(2, 2)].astype(jnp.float32)
            ).astype(jnp.bfloat16)
            rd2[g] = mk(sb2, rb2, g, 2, axes[1])
            rd2[g].start()
        for g in ORDER:
            acc, rb1 = sc[g][0], sc[g][2]
            m1 = mbit[GROUPS[g][0][0]]
            rd1b[g].wait()
            for j in (0, 1):
                acc[j] = partial(slot_chunk(g, m1, j), g) + rb1[j].astype(jnp.float32)
        for g in ORDER:
            axes, w = GROUPS[g]
            acc, rb2, sb3, rb3 = sc[g][0], sc[g][4], sc[g][5], sc[g][6]
            rd2[g].wait()
            sb3[:, :, :] = (
                acc[pl.ds(1, 1)] + rb2[pl.ds(1, 1)].astype(jnp.float32)
            ).astype(jnp.bfloat16)
            rd3[g] = mk(sb3, rb3, g, 3, axes[2])
            rd3[g].start()
        scale = sx_ref[0] * sw_ref[0]
        for g in ORDER:
            acc, rb2, rb3 = sc[g][0], sc[g][4], sc[g][6]
            rd3[g].wait()
            final = acc[0] + rb2[0].astype(jnp.float32) + rb3[0].astype(jnp.float32)
            y = final * scale
            out_ref[:, col0s[g]:col0s[g] + GROUPS[g][1]] = y * jax.nn.sigmoid(y)

    scratch = []
    for (_, w) in GROUPS:
        scratch += [
            pltpu.VMEM((4, m_per, w), jnp.float32),
            pltpu.VMEM((4, m_per, w), jnp.bfloat16),
            pltpu.VMEM((4, m_per, w), jnp.bfloat16),
            pltpu.VMEM((2, m_per, w), jnp.bfloat16),
            pltpu.VMEM((2, m_per, w), jnp.bfloat16),
            pltpu.VMEM((1, m_per, w), jnp.bfloat16),
            pltpu.VMEM((1, m_per, w), jnp.bfloat16),
            pltpu.SemaphoreType.DMA((4,)),
            pltpu.SemaphoreType.DMA((4,)),
        ]

    return pl.pallas_call(
        body,
        out_shape=jax.ShapeDtypeStruct((m_per, n), jnp.float32),
        in_specs=[
            pl.BlockSpec(memory_space=pltpu.VMEM),
            pl.BlockSpec(memory_space=pltpu.VMEM),
            pl.BlockSpec(memory_space=pltpu.SMEM),
            pl.BlockSpec(memory_space=pltpu.SMEM),
        ],
        out_specs=pl.BlockSpec(memory_space=pltpu.VMEM),
        scratch_shapes=scratch,
        compiler_params=pltpu.CompilerParams(
            collective_id=0,
            vmem_limit_bytes=110 * 1024 * 1024,
        ),
    )(x, w_mat, scale_x, scale_w)


# device time: 73029 ns/iter; 1.2327x vs baseline; 1.0365x over previous
import jax
import jax.numpy as jnp
from jax import lax
from jax.experimental import pallas as pl
from jax.experimental.pallas import tpu as pltpu

N_DEV = 8

GROUPS = (
    ((2, 1, 0), 640),
    ((1, 0, 2), 640),
    ((0, 2, 1), 768),
)
ORDER = sorted(range(len(GROUPS)), key=lambda g: -GROUPS[g][1])


def kernel(x, w_mat, scale_x, scale_w):
    m, k_per = x.shape
    _, n = w_mat.shape
    m_per = m // N_DEV
    assert sum(w for _, w in GROUPS) == n

    def body(x_ref, w_ref, sx_ref, sw_ref, out_ref, *scratch):
        sc = [scratch[9 * g:9 * (g + 1)] for g in range(len(GROUPS))]

        my = lax.axis_index("i")
        p4 = lax.rem(my, 4)
        mbit = (
            lax.rem(lax.rem(p4, 2) + p4 // 2, 2),
            p4 // 2,
            my // 4,
        )

        def pos_from_bits(bx, by, bz):
            return 4 * bz + bx + by * (3 - 2 * bx)

        def partner(axis):
            b = list(mbit)
            b[axis] = 1 - b[axis]
            return pos_from_bits(*b)

        def bxor(j, mb):
            return 1 - mb if j else mb

        barrier = pltpu.get_barrier_semaphore()
        for a in range(3):
            pl.semaphore_signal(
                barrier, inc=1, device_id=(partner(a),),
                device_id_type=pl.DeviceIdType.MESH,
            )
        pl.semaphore_wait(barrier, 3)

        col0s = []
        c0 = 0
        for (_, w) in GROUPS:
            col0s.append(c0)
            c0 += w

        def partial(c, g):
            xb = x_ref[pl.ds(c * m_per, m_per), :].astype(jnp.bfloat16)
            wb = w_ref[:, col0s[g]:col0s[g] + GROUPS[g][1]].astype(jnp.bfloat16)
            return jnp.dot(xb, wb, preferred_element_type=jnp.float32)

        def mk(src, dst, g, stage, axis):
            ss, rs = sc[g][7], sc[g][8]
            return pltpu.make_async_remote_copy(
                src_ref=src, dst_ref=dst,
                send_sem=ss.at[stage], recv_sem=rs.at[stage],
                device_id=(partner(axis),),
                device_id_type=pl.DeviceIdType.MESH,
            )

        def slot_chunk(g, b1, j):
            axes = GROUPS[g][0]
            bits = [None, None, None]
            bits[axes[0]] = b1
            bits[axes[1]] = bxor(j // 2, mbit[axes[1]])
            bits[axes[2]] = bxor(j % 2, mbit[axes[2]])
            return pos_from_bits(bits[0], bits[1], bits[2])

        rd1 = {}
        rd2a, rd2b, rd3 = {}, {}, {}
        for g in ORDER:
            axes, w = GROUPS[g]
            sb1, rb1 = sc[g][1], sc[g][2]
            m1 = mbit[axes[0]]
            for j in (2, 3):
                sb1[j] = partial(slot_chunk(g, 1 - m1, j), g).astype(jnp.bfloat16)
                rd1[g, j] = mk(sb1.at[pl.ds(j, 1)], rb1.at[pl.ds(j, 1)], g, j, axes[0])
                rd1[g, j].start()
        for g in ORDER:
            axes, w = GROUPS[g]
            sb1, rb1 = sc[g][1], sc[g][2]
            m1 = mbit[axes[0]]
            for j in (0, 1):
                sb1[j] = partial(slot_chunk(g, 1 - m1, j), g).astype(jnp.bfloat16)
                rd1[g, j] = mk(sb1.at[pl.ds(j, 1)], rb1.at[pl.ds(j, 1)], g, j, axes[0])
                rd1[g, j].start()
        for g in ORDER:
            acc = sc[g][0]
            m1 = mbit[GROUPS[g][0][0]]
            for j in (2, 3):
                acc[j] = partial(slot_chunk(g, m1, j), g)
        for g in ORDER:
            axes, w = GROUPS[g]
            acc, rb1, sb2, rb2 = sc[g][0], sc[g][2], sc[g][3], sc[g][4]
            rd1[g, 3].wait()
            sb2[1] = (acc[3] + rb1[3].astype(jnp.float32)).astype(jnp.bfloat16)
            rd2a[g] = mk(sb2.at[pl.ds(1, 1)], rb2.at[pl.ds(1, 1)], g, 4, axes[1])
            rd2a[g].start()
            rd1[g, 2].wait()
            sb2[0] = (acc[2] + rb1[2].astype(jnp.float32)).astype(jnp.bfloat16)
            rd2b[g] = mk(sb2.at[pl.ds(0, 1)], rb2.at[pl.ds(0, 1)], g, 5, axes[1])
            rd2b[g].start()
        for g in ORDER:
            acc, rb1 = sc[g][0], sc[g][2]
            m1 = mbit[GROUPS[g][0][0]]
            for j in (1, 0):
                rd1[g, j].wait()
                acc[j] = partial(slot_chunk(g, m1, j), g) + rb1[j].astype(jnp.float32)
        for g in ORDER:
            axes, w = GROUPS[g]
            acc, rb2, sb3, rb3 = sc[g][0], sc[g][4], sc[g][5], sc[g][6]
            rd2a[g].wait()
            sb3[:, :, :] = (
                acc[pl.ds(1, 1)] + rb2[pl.ds(1, 1)].astype(jnp.float32)
            ).astype(jnp.bfloat16)
            rd3[g] = mk(sb3, rb3, g, 6, axes[2])
            rd3[g].start()
        scale = sx_ref[0] * sw_ref[0]
        for g in ORDER:
            acc, rb2, rb3 = sc[g][0], sc[g][4], sc[g][6]
            rd2b[g].wait()
            rd3[g].wait()
            final = acc[0] + rb2[0].astype(jnp.float32) + rb3[0].astype(jnp.float32)
            y = final * scale
            out_ref[:, col0s[g]:col0s[g] + GROUPS[g][1]] = y * jax.nn.sigmoid(y)

    scratch = []
    for (_, w) in GROUPS:
        scratch += [
            pltpu.VMEM((4, m_per, w), jnp.float32),
            pltpu.VMEM((4, m_per, w), jnp.bfloat16),
            pltpu.VMEM((4, m_per, w), jnp.bfloat16),
            pltpu.VMEM((2, m_per, w), jnp.bfloat16),
            pltpu.VMEM((2, m_per, w), jnp.bfloat16),
            pltpu.VMEM((1, m_per, w), jnp.bfloat16),
            pltpu.VMEM((1, m_per, w), jnp.bfloat16),
            pltpu.SemaphoreType.DMA((7,)),
            pltpu.SemaphoreType.DMA((7,)),
        ]

    return pl.pallas_call(
        body,
        out_shape=jax.ShapeDtypeStruct((m_per, n), jnp.float32),
        in_specs=[
            pl.BlockSpec(memory_space=pltpu.VMEM),
            pl.BlockSpec(memory_space=pltpu.VMEM),
            pl.BlockSpec(memory_space=pltpu.SMEM),
            pl.BlockSpec(memory_space=pltpu.SMEM),
        ],
        out_specs=pl.BlockSpec(memory_space=pltpu.VMEM),
        scratch_shapes=scratch,
        compiler_params=pltpu.CompilerParams(
            collective_id=0,
            vmem_limit_bytes=110 * 1024 * 1024,
        ),
    )(x, w_mat, scale_x, scale_w)
